# baseline (speedup 1.0000x reference)
import functools

import jax
import jax.numpy as jnp
from jax import lax
from jax.experimental import pallas as pl
from jax.experimental.pallas import tpu as pltpu
from jax.experimental.pallas import tpu_sc as plsc

B = 16384
D = 64
NC = 2
NS = 16
NW = NC * NS
BW = B // NW

_mesh = plsc.VectorSubcoreMesh(core_axis_name="c", subcore_axis_name="s")


@functools.partial(
    pl.kernel,
    out_type=(
        jax.ShapeDtypeStruct((B, D), jnp.float32),
        jax.ShapeDtypeStruct((B, D), jnp.float32),
        jax.ShapeDtypeStruct((B, D), jnp.float32),
    ),
    mesh=_mesh,
    scratch_types=[
        pltpu.VMEM((BW,), jnp.int32),
        pltpu.VMEM((BW,), jnp.int32),
        pltpu.VMEM((BW,), jnp.int32),
        pltpu.SemaphoreType.DMA,
        pltpu.SemaphoreType.DMA,
    ],
)
def _sc_gather(u_idx_hbm, p_idx_hbm, n_idx_hbm, user_hbm, item_hbm,
               u_out, p_out, n_out,
               iu_v, ip_v, in_v, isem, sem):
    wid = lax.axis_index("s") * NC + lax.axis_index("c")
    base = wid * BW
    sl = pl.ds(base, BW)
    pltpu.async_copy(u_idx_hbm.at[sl], iu_v, isem).wait()
    pltpu.async_copy(p_idx_hbm.at[sl], ip_v, isem).wait()
    pltpu.async_copy(n_idx_hbm.at[sl], in_v, isem).wait()

    @pl.loop(0, BW, step=16)
    def _(k):
        vu = iu_v[pl.ds(k, 16)]
        vp = ip_v[pl.ds(k, 16)]
        vn = in_v[pl.ds(k, 16)]
        for j in range(16):
            b = base + k + j
            pltpu.async_copy(user_hbm.at[vu[j]], u_out.at[b], sem)
            pltpu.async_copy(item_hbm.at[vp[j]], p_out.at[b], sem)
            pltpu.async_copy(item_hbm.at[vn[j]], n_out.at[b], sem)

    @pl.loop(0, 3 * BW, step=32)
    def _(k):
        pltpu.make_async_copy(
            user_hbm.at[pl.ds(0, 32)], u_out.at[pl.ds(0, 32)], sem).wait()


_TC_BLK = 2048


def _tc_dist_body(u_ref, i_ref, j_ref, pos_ref, neg_ref):
    u = u_ref[...]
    i = i_ref[...]
    j = j_ref[...]
    ssq_u = jnp.sum(u * u, axis=1, keepdims=True)
    ssq_i = jnp.sum(i * i, axis=1, keepdims=True)
    ssq_j = jnp.sum(j * j, axis=1, keepdims=True)
    dot_i = jnp.sum(u * i, axis=1, keepdims=True)
    dot_j = jnp.sum(u * j, axis=1, keepdims=True)
    mu = jnp.maximum(ssq_u, 1.0)
    mi = jnp.maximum(ssq_i, 1.0)
    mj = jnp.maximum(ssq_j, 1.0)
    pos_ref[...] = ssq_u / mu + ssq_i / mi - 2.0 * dot_i * lax.rsqrt(mu * mi)
    neg_ref[...] = ssq_u / mu + ssq_j / mj - 2.0 * dot_j * lax.rsqrt(mu * mj)


def _tc_dist(u_rows, p_rows, n_rows):
    row_spec = pl.BlockSpec((_TC_BLK, D), lambda b: (b, 0))
    out_spec = pl.BlockSpec((_TC_BLK, 1), lambda b: (b, 0))
    return pl.pallas_call(
        _tc_dist_body,
        grid=(B // _TC_BLK,),
        in_specs=[row_spec, row_spec, row_spec],
        out_specs=[out_spec, out_spec],
        out_shape=[
            jax.ShapeDtypeStruct((B, 1), jnp.float32),
            jax.ShapeDtypeStruct((B, 1), jnp.float32),
        ],
    )(u_rows, p_rows, n_rows)


def kernel(batch_user, batch_pos_item, batch_neg_item, user_emb, item_emb):
    u_rows, p_rows, n_rows = _sc_gather(
        batch_user, batch_pos_item, batch_neg_item, user_emb, item_emb)
    pos, neg = _tc_dist(u_rows, p_rows, n_rows)
    return (pos, neg)


# speedup vs baseline: 1.6088x; 1.6088x over previous
import functools

import jax
import jax.numpy as jnp
from jax import lax
from jax.experimental import pallas as pl
from jax.experimental.pallas import tpu as pltpu
from jax.experimental.pallas import tpu_sc as plsc

B = 16384
D = 64
NC = 2
NS = 16
NW = NC * NS
BW = B // NW

_mesh = plsc.VectorSubcoreMesh(core_axis_name="c", subcore_axis_name="s")


@functools.partial(
    pl.kernel,
    out_type=(
        jax.ShapeDtypeStruct((B, 2 * D), jnp.float32),
        jax.ShapeDtypeStruct((B, 2 * D), jnp.float32),
        jax.ShapeDtypeStruct((B, 2 * D), jnp.float32),
    ),
    mesh=_mesh,
    scratch_types=[
        pltpu.VMEM((BW,), jnp.int32),
        pltpu.VMEM((BW, 2 * D), jnp.float32),
        pltpu.SemaphoreType.DMA,
        pltpu.SemaphoreType.DMA,
    ],
)
def _sc_gather(u_idx_hbm, p_idx_hbm, n_idx_hbm, user_p_hbm, item_p_hbm,
               u_out, p_out, n_out, tix_v, pairs_v, isem, gsem):
    wid = lax.axis_index("s") * NC + lax.axis_index("c")
    base = wid * BW
    sl = pl.ds(base, BW)

    def gather_one(idx_hbm, tab, out):
        pltpu.async_copy(idx_hbm.at[sl], tix_v, isem).wait()

        @pl.loop(0, BW, step=16)
        def _(k):
            tix_v[pl.ds(k, 16)] = lax.shift_right_logical(tix_v[pl.ds(k, 16)], 1)

        pltpu.async_copy(tab.at[tix_v], pairs_v, gsem).wait()
        pltpu.sync_copy(pairs_v, out.at[sl])

    gather_one(u_idx_hbm, user_p_hbm, u_out)
    gather_one(p_idx_hbm, item_p_hbm, p_out)
    gather_one(n_idx_hbm, item_p_hbm, n_out)


_TC_BLK = 2048


def _tc_dist_body(bu_ref, bp_ref, bn_ref, u_ref, i_ref, j_ref,
                  pos_ref, neg_ref):
    def pick(pair_ref, idx_ref):
        x = pair_ref[...]
        odd = lax.bitwise_and(idx_ref[...], 1) == 1
        return jnp.where(odd, x[:, D:], x[:, :D])

    u = pick(u_ref, bu_ref)
    i = pick(i_ref, bp_ref)
    j = pick(j_ref, bn_ref)
    ssq_u = jnp.sum(u * u, axis=1, keepdims=True)
    ssq_i = jnp.sum(i * i, axis=1, keepdims=True)
    ssq_j = jnp.sum(j * j, axis=1, keepdims=True)
    dot_i = jnp.sum(u * i, axis=1, keepdims=True)
    dot_j = jnp.sum(u * j, axis=1, keepdims=True)
    mu = jnp.maximum(ssq_u, 1.0)
    mi = jnp.maximum(ssq_i, 1.0)
    mj = jnp.maximum(ssq_j, 1.0)
    pos_ref[...] = ssq_u / mu + ssq_i / mi - 2.0 * dot_i * lax.rsqrt(mu * mi)
    neg_ref[...] = ssq_u / mu + ssq_j / mj - 2.0 * dot_j * lax.rsqrt(mu * mj)


def _tc_dist(bu, bp, bn, u_pairs, p_pairs, n_pairs):
    pair_spec = pl.BlockSpec((_TC_BLK, 2 * D), lambda b: (b, 0))
    idx_spec = pl.BlockSpec((_TC_BLK, 1), lambda b: (b, 0))
    out_spec = pl.BlockSpec((_TC_BLK, 1), lambda b: (b, 0))
    return pl.pallas_call(
        _tc_dist_body,
        grid=(B // _TC_BLK,),
        in_specs=[idx_spec, idx_spec, idx_spec, pair_spec, pair_spec, pair_spec],
        out_specs=[out_spec, out_spec],
        out_shape=[
            jax.ShapeDtypeStruct((B, 1), jnp.float32),
            jax.ShapeDtypeStruct((B, 1), jnp.float32),
        ],
    )(bu, bp, bn, u_pairs, p_pairs, n_pairs)


def kernel(batch_user, batch_pos_item, batch_neg_item, user_emb, item_emb):
    user_p = user_emb.reshape(user_emb.shape[0] // 2, 2 * D)
    item_p = item_emb.reshape(item_emb.shape[0] // 2, 2 * D)
    u_pairs, p_pairs, n_pairs = _sc_gather(
        batch_user, batch_pos_item, batch_neg_item, user_p, item_p)
    pos, neg = _tc_dist(
        batch_user.reshape(B, 1), batch_pos_item.reshape(B, 1),
        batch_neg_item.reshape(B, 1), u_pairs, p_pairs, n_pairs)
    return (pos, neg)
